# baseline (speedup 1.0000x reference)
import functools

import jax
import jax.numpy as jnp
from jax.experimental import pallas as pl
from jax.experimental.pallas import tpu as pltpu

_E = 8
_K = 2
_D = 1024
_H = 1024
_F = 512
_O = 1024
_N = 2048
_T = 512
_P = _N * _K + _E * _T
_V = 9728


def _ln(t, g, b):
    m = jnp.mean(t, axis=-1, keepdims=True)
    v = jnp.mean((t - m) ** 2, axis=-1, keepdims=True)
    return (t - m) * jax.lax.rsqrt(v + 1e-5) * g + b


def _gelu(t):
    return 0.5 * t * (1.0 + jax.lax.erf(t * 0.7071067811865476))


def _expert_body(gid_ref, x_ref, w1r_ref, w2_ref, w3_ref, w4_ref, vec_ref,
                 out_ref):
    x = x_ref[...]
    v = vec_ref[0, 0]
    hr = jnp.dot(x, w1r_ref[0], preferred_element_type=jnp.float32)
    h = hr[:, :_H] + v[0:1024]
    r = hr[:, _H:] + v[1024:2048]
    h = _gelu(_ln(h, v[2048:3072], v[3072:4096])) + r
    h = h.astype(jnp.bfloat16)
    h = jnp.dot(h, w2_ref[0], preferred_element_type=jnp.float32) + v[4096:5120]
    h = _gelu(_ln(h, v[5120:6144], v[6144:7168])).astype(jnp.bfloat16)
    h = jnp.dot(h, w3_ref[0], preferred_element_type=jnp.float32) + v[7168:7680]
    h = _gelu(_ln(h, v[7680:8192], v[8192:8704])).astype(jnp.bfloat16)
    o = jnp.dot(h, w4_ref[0], preferred_element_type=jnp.float32) + v[8704:9728]
    out_ref[...] = o.astype(jnp.bfloat16)


def _grouped_mlp(gid, xs, w1r, w2, w3, w4, vec):
    n_tiles = _P // _T

    def row_spec(cols):
        return pl.BlockSpec((_T, cols), lambda i, g: (i, 0))

    def w3_spec(r, c):
        return pl.BlockSpec((1, r, c), lambda i, g: (g[i], 0, 0))

    grid_spec = pltpu.PrefetchScalarGridSpec(
        num_scalar_prefetch=1,
        grid=(n_tiles,),
        in_specs=[
            row_spec(_D),
            w3_spec(_D, 2 * _H),
            w3_spec(_H, _H),
            w3_spec(_H, _F),
            w3_spec(_F, _O),
            pl.BlockSpec((1, 1, _V), lambda i, g: (g[i], 0, 0)),
        ],
        out_specs=row_spec(_O),
    )
    return pl.pallas_call(
        _expert_body,
        grid_spec=grid_spec,
        out_shape=jax.ShapeDtypeStruct((_P, _O), jnp.bfloat16),
    )(gid, xs, w1r, w2, w3, w4, vec)


def kernel(x, gate_W, gate_b, fc1_W, fc1_b, ln1_g, ln1_b, res_W, res_b,
           fc2_W, fc2_b, ln2_g, ln2_b, fc3_W, fc3_b, ln3_g, ln3_b,
           fc4_W, fc4_b):
    logits = x @ gate_W + gate_b
    probs = jax.nn.softmax(logits, axis=-1)
    topv, topi = jax.lax.top_k(probs, _K)
    wn = topv / (jnp.sum(topv, axis=-1, keepdims=True) + 1e-9)

    e_flat = topi.reshape(-1).astype(jnp.int32)
    onehot = (e_flat[:, None] == jnp.arange(_E, dtype=jnp.int32)[None, :])
    csum = jnp.cumsum(onehot.astype(jnp.int32), axis=0)
    rank = jnp.take_along_axis(csum, e_flat[:, None], axis=1)[:, 0] - 1
    counts = csum[-1]

    padded = ((counts + _T - 1) // _T) * _T
    pad_end = jnp.cumsum(padded)
    pad_off = pad_end - padded
    dest = (pad_off[e_flat] + rank).astype(jnp.int32)

    tok = (jnp.arange(_N * _K, dtype=jnp.int32) // _K)
    gather_idx = jnp.zeros((_P,), jnp.int32).at[dest].set(tok)
    pos = dest.reshape(_N, _K)

    tile_start = jnp.arange(_P // _T, dtype=jnp.int32) * _T
    gid = jnp.searchsorted(pad_end, tile_start, side='right')
    gid = jnp.minimum(gid, _E - 1).astype(jnp.int32)

    xs = jnp.take(x.astype(jnp.bfloat16), gather_idx, axis=0)

    w1r = jnp.concatenate([fc1_W, res_W], axis=2).astype(jnp.bfloat16)
    vec = jnp.concatenate(
        [fc1_b, res_b, ln1_g, ln1_b, fc2_b, ln2_g, ln2_b,
         fc3_b, ln3_g, ln3_b, fc4_b], axis=1)[:, None, :]

    out_sorted = _grouped_mlp(
        gid, xs, w1r, fc2_W.astype(jnp.bfloat16), fc3_W.astype(jnp.bfloat16),
        fc4_W.astype(jnp.bfloat16), vec)

    y = (jnp.take(out_sorted, pos[:, 0], axis=0) * wn[:, 0:1]
         + jnp.take(out_sorted, pos[:, 1], axis=0) * wn[:, 1:2])
    return y


# speedup vs baseline: 1.0905x; 1.0905x over previous
import functools

import jax
import jax.numpy as jnp
from jax.experimental import pallas as pl
from jax.experimental.pallas import tpu as pltpu

_E = 8
_K = 2
_D = 1024
_H = 1024
_F = 512
_O = 1024
_N = 2048
_T = 256
_P = _N * _K + _E * _T
_V = 9728


def _ln(t, g, b):
    m = jnp.mean(t, axis=-1, keepdims=True)
    v = jnp.mean((t - m) ** 2, axis=-1, keepdims=True)
    return (t - m) * jax.lax.rsqrt(v + 1e-5) * g + b


def _gelu(t):
    return 0.5 * t * (1.0 + jax.lax.erf(t * 0.7071067811865476))


def _expert_body(gid_ref, act_ref, x_ref, w1r_ref, w2_ref, w3_ref, w4_ref,
                 vec_ref, out_ref):
    @pl.when(act_ref[pl.program_id(0)] > 0)
    def _():
        x = x_ref[...]
        v = vec_ref[0, 0]
        hr = jnp.dot(x, w1r_ref[0], preferred_element_type=jnp.float32)
        h = hr[:, :_H] + v[0:1024]
        r = hr[:, _H:] + v[1024:2048]
        h = _gelu(_ln(h, v[2048:3072], v[3072:4096])) + r
        h = h.astype(jnp.bfloat16)
        h = jnp.dot(h, w2_ref[0], preferred_element_type=jnp.float32) + v[4096:5120]
        h = _gelu(_ln(h, v[5120:6144], v[6144:7168])).astype(jnp.bfloat16)
        h = jnp.dot(h, w3_ref[0], preferred_element_type=jnp.float32) + v[7168:7680]
        h = _gelu(_ln(h, v[7680:8192], v[8192:8704])).astype(jnp.bfloat16)
        o = jnp.dot(h, w4_ref[0], preferred_element_type=jnp.float32) + v[8704:9728]
        out_ref[...] = o.astype(jnp.bfloat16)


def _grouped_mlp(gid, act, xs, w1r, w2, w3, w4, vec):
    n_tiles = _P // _T

    def row_spec(cols):
        return pl.BlockSpec((_T, cols), lambda i, g, a: (i, 0))

    def w3_spec(r, c):
        return pl.BlockSpec((1, r, c), lambda i, g, a: (g[i], 0, 0))

    grid_spec = pltpu.PrefetchScalarGridSpec(
        num_scalar_prefetch=2,
        grid=(n_tiles,),
        in_specs=[
            row_spec(_D),
            w3_spec(_D, 2 * _H),
            w3_spec(_H, _H),
            w3_spec(_H, _F),
            w3_spec(_F, _O),
            pl.BlockSpec((1, 1, _V), lambda i, g, a: (g[i], 0, 0)),
        ],
        out_specs=row_spec(_O),
    )
    return pl.pallas_call(
        _expert_body,
        grid_spec=grid_spec,
        out_shape=jax.ShapeDtypeStruct((_P, _O), jnp.bfloat16),
    )(gid, act, xs, w1r, w2, w3, w4, vec)


def kernel(x, gate_W, gate_b, fc1_W, fc1_b, ln1_g, ln1_b, res_W, res_b,
           fc2_W, fc2_b, ln2_g, ln2_b, fc3_W, fc3_b, ln3_g, ln3_b,
           fc4_W, fc4_b):
    logits = x @ gate_W + gate_b
    probs = jax.nn.softmax(logits, axis=-1)
    topv, topi = jax.lax.top_k(probs, _K)
    wn = topv / (jnp.sum(topv, axis=-1, keepdims=True) + 1e-9)

    e_flat = topi.reshape(-1).astype(jnp.int32)
    onehot = (e_flat[:, None] == jnp.arange(_E, dtype=jnp.int32)[None, :])
    csum = jnp.cumsum(onehot.astype(jnp.int32), axis=0)
    rank = jnp.take_along_axis(csum, e_flat[:, None], axis=1)[:, 0] - 1
    counts = csum[-1]

    padded = ((counts + _T - 1) // _T) * _T
    pad_end = jnp.cumsum(padded)
    pad_off = pad_end - padded
    dest = (pad_off[e_flat] + rank).astype(jnp.int32)

    tok = (jnp.arange(_N * _K, dtype=jnp.int32) // _K)
    gather_idx = jnp.zeros((_P,), jnp.int32).at[dest].set(tok)
    pos = dest.reshape(_N, _K)

    tile_start = jnp.arange(_P // _T, dtype=jnp.int32) * _T
    gid = jnp.searchsorted(pad_end, tile_start, side='right')
    gid = jnp.minimum(gid, _E - 1).astype(jnp.int32)
    act = (tile_start < pad_end[_E - 1]).astype(jnp.int32)

    xs = jnp.take(x.astype(jnp.bfloat16), gather_idx, axis=0)

    w1r = jnp.concatenate([fc1_W, res_W], axis=2).astype(jnp.bfloat16)
    vec = jnp.concatenate(
        [fc1_b, res_b, ln1_g, ln1_b, fc2_b, ln2_g, ln2_b,
         fc3_b, ln3_g, ln3_b, fc4_b], axis=1)[:, None, :]

    out_sorted = _grouped_mlp(
        gid, act, xs, w1r, fc2_W.astype(jnp.bfloat16),
        fc3_W.astype(jnp.bfloat16), fc4_W.astype(jnp.bfloat16), vec)

    y = (jnp.take(out_sorted, pos[:, 0], axis=0) * wn[:, 0:1]
         + jnp.take(out_sorted, pos[:, 1], axis=0) * wn[:, 1:2])
    return y


# speedup vs baseline: 1.0989x; 1.0076x over previous
import functools

import jax
import jax.numpy as jnp
from jax.experimental import pallas as pl
from jax.experimental.pallas import tpu as pltpu

_E = 8
_K = 2
_D = 1024
_H = 1024
_F = 512
_O = 1024
_N = 2048
_T = 256
_P = _N * _K + _E * _T
_V = 9728


def _ln(t, g, b):
    m = jnp.mean(t, axis=-1, keepdims=True)
    v = jnp.mean((t - m) ** 2, axis=-1, keepdims=True)
    return (t - m) * jax.lax.rsqrt(v + 1e-5) * g + b


def _gelu(t):
    return 0.5 * t * (1.0 + jax.lax.erf(t * 0.7071067811865476))


def _expert_body(gid_ref, act_ref, x_ref, w1r_ref, w2_ref, w3_ref, w4_ref,
                 vec_ref, out_ref):
    @pl.when(act_ref[pl.program_id(0)] > 0)
    def _():
        x = x_ref[...]
        v = vec_ref[0, 0]
        hr = jnp.dot(x, w1r_ref[0], preferred_element_type=jnp.float32)
        h = hr[:, :_H] + v[0:1024]
        r = hr[:, _H:] + v[1024:2048]
        h = _gelu(_ln(h, v[2048:3072], v[3072:4096])) + r
        h = h.astype(jnp.bfloat16)
        h = jnp.dot(h, w2_ref[0], preferred_element_type=jnp.float32) + v[4096:5120]
        h = _gelu(_ln(h, v[5120:6144], v[6144:7168])).astype(jnp.bfloat16)
        h = jnp.dot(h, w3_ref[0], preferred_element_type=jnp.float32) + v[7168:7680]
        h = _gelu(_ln(h, v[7680:8192], v[8192:8704])).astype(jnp.bfloat16)
        o = jnp.dot(h, w4_ref[0], preferred_element_type=jnp.float32) + v[8704:9728]
        out_ref[...] = o.astype(jnp.bfloat16)


def _grouped_mlp(gid, act, xs, w1r, w2, w3, w4, vec):
    n_tiles = _P // _T

    def row_spec(cols):
        return pl.BlockSpec((_T, cols), lambda i, g, a: (i, 0))

    def w3_spec(r, c):
        return pl.BlockSpec((1, r, c), lambda i, g, a: (g[i], 0, 0))

    grid_spec = pltpu.PrefetchScalarGridSpec(
        num_scalar_prefetch=2,
        grid=(n_tiles,),
        in_specs=[
            row_spec(_D),
            w3_spec(_D, 2 * _H),
            w3_spec(_H, _H),
            w3_spec(_H, _F),
            w3_spec(_F, _O),
            pl.BlockSpec((1, 1, _V), lambda i, g, a: (g[i], 0, 0)),
        ],
        out_specs=row_spec(_O),
    )
    return pl.pallas_call(
        _expert_body,
        grid_spec=grid_spec,
        out_shape=jax.ShapeDtypeStruct((_P, _O), jnp.bfloat16),
    )(gid, act, xs, w1r, w2, w3, w4, vec)


def kernel(x, gate_W, gate_b, fc1_W, fc1_b, ln1_g, ln1_b, res_W, res_b,
           fc2_W, fc2_b, ln2_g, ln2_b, fc3_W, fc3_b, ln3_g, ln3_b,
           fc4_W, fc4_b):
    logits = x @ gate_W + gate_b
    probs = jax.nn.softmax(logits, axis=-1)
    lane = jnp.arange(_E, dtype=jnp.int32)[None, :]
    i1 = jnp.argmax(probs, axis=-1).astype(jnp.int32)
    v1 = jnp.max(probs, axis=-1)
    p2 = jnp.where(lane == i1[:, None], -1.0, probs)
    i2 = jnp.argmax(p2, axis=-1).astype(jnp.int32)
    v2 = jnp.max(p2, axis=-1)
    s = v1 + v2 + 1e-9
    wn = jnp.stack([v1 / s, v2 / s], axis=1)

    e_flat = jnp.stack([i1, i2], axis=1).reshape(-1)
    onehot = (e_flat[:, None] == lane).astype(jnp.float32)
    ohb = onehot.reshape(_N * _K // 128, 128, _E)
    r_iota = jnp.arange(128, dtype=jnp.int32)
    tril_s = (r_iota[:, None] > r_iota[None, :]).astype(jnp.float32)
    pre = jnp.einsum('rs,bse->bre', tril_s, ohb)
    blocksum = jnp.sum(ohb, axis=1)
    blockpre = jnp.cumsum(blocksum, axis=0) - blocksum
    rank_full = (blockpre[:, None, :] + pre).reshape(_N * _K, _E)
    rank = jnp.take_along_axis(rank_full, e_flat[:, None], axis=1)[:, 0]
    rank = rank.astype(jnp.int32)
    counts = jnp.sum(blocksum, axis=0).astype(jnp.int32)

    padded = ((counts + _T - 1) // _T) * _T
    pad_end = jnp.cumsum(padded)
    pad_off = pad_end - padded
    dest = (pad_off[e_flat] + rank).astype(jnp.int32)

    tok = (jnp.arange(_N * _K, dtype=jnp.int32) // _K)
    gather_idx = jnp.zeros((_P,), jnp.int32).at[dest].set(tok)
    pos = dest.reshape(_N, _K)

    tile_start = jnp.arange(_P // _T, dtype=jnp.int32) * _T
    gid = jnp.searchsorted(pad_end, tile_start, side='right')
    gid = jnp.minimum(gid, _E - 1).astype(jnp.int32)
    act = (tile_start < pad_end[_E - 1]).astype(jnp.int32)

    xs = jnp.take(x.astype(jnp.bfloat16), gather_idx, axis=0)

    w1r = jnp.concatenate([fc1_W, res_W], axis=2).astype(jnp.bfloat16)
    vec = jnp.concatenate(
        [fc1_b, res_b, ln1_g, ln1_b, fc2_b, ln2_g, ln2_b,
         fc3_b, ln3_g, ln3_b, fc4_b], axis=1)[:, None, :]

    out_sorted = _grouped_mlp(
        gid, act, xs, w1r, fc2_W.astype(jnp.bfloat16),
        fc3_W.astype(jnp.bfloat16), fc4_W.astype(jnp.bfloat16), vec)

    y = (jnp.take(out_sorted, pos[:, 0], axis=0) * wn[:, 0:1]
         + jnp.take(out_sorted, pos[:, 1], axis=0) * wn[:, 1:2])
    return y


# speedup vs baseline: 1.4120x; 1.2849x over previous
import functools

import jax
import jax.numpy as jnp
from jax.experimental import pallas as pl
from jax.experimental.pallas import tpu as pltpu

_E = 8
_K = 2
_D = 1024
_H = 1024
_F = 512
_O = 1024
_N = 2048
_T = 256
_P = _N * _K + _E * _T
_V = 9728


def _ln(t, g, b):
    m = jnp.mean(t, axis=-1, keepdims=True)
    v = jnp.mean((t - m) ** 2, axis=-1, keepdims=True)
    return (t - m) * jax.lax.rsqrt(v + 1e-5) * g + b


def _gelu(t):
    return 0.5 * t * (1.0 + jax.lax.erf(t * 0.7071067811865476))


def _expert_body(gid_ref, act_ref, x_ref, w1r_ref, res_ref, w2_ref, w3_ref,
                 w4_ref, vec_ref, out_ref):
    @pl.when(act_ref[pl.program_id(0)] > 0)
    def _():
        x = x_ref[...]
        v = vec_ref[0, 0]
        b16 = lambda w: w.astype(jnp.bfloat16)
        h = jnp.dot(x, b16(w1r_ref[0]), preferred_element_type=jnp.float32)
        h = h + v[0:1024]
        r = jnp.dot(x, b16(res_ref[0]), preferred_element_type=jnp.float32)
        r = r + v[1024:2048]
        h = _gelu(_ln(h, v[2048:3072], v[3072:4096])) + r
        h = h.astype(jnp.bfloat16)
        h = jnp.dot(h, b16(w2_ref[0]), preferred_element_type=jnp.float32) + v[4096:5120]
        h = _gelu(_ln(h, v[5120:6144], v[6144:7168])).astype(jnp.bfloat16)
        h = jnp.dot(h, b16(w3_ref[0]), preferred_element_type=jnp.float32) + v[7168:7680]
        h = _gelu(_ln(h, v[7680:8192], v[8192:8704])).astype(jnp.bfloat16)
        o = jnp.dot(h, b16(w4_ref[0]), preferred_element_type=jnp.float32) + v[8704:9728]
        out_ref[...] = o.astype(jnp.bfloat16)


def _grouped_mlp(gid, act, xs, w1r, res, w2, w3, w4, vec):
    n_tiles = _P // _T

    def row_spec(cols):
        return pl.BlockSpec((_T, cols), lambda i, g, a: (i, 0))

    def w3_spec(r, c):
        return pl.BlockSpec((1, r, c), lambda i, g, a: (g[i], 0, 0))

    grid_spec = pltpu.PrefetchScalarGridSpec(
        num_scalar_prefetch=2,
        grid=(n_tiles,),
        in_specs=[
            row_spec(_D),
            w3_spec(_D, _H),
            w3_spec(_D, _H),
            w3_spec(_H, _H),
            w3_spec(_H, _F),
            w3_spec(_F, _O),
            pl.BlockSpec((1, 1, _V), lambda i, g, a: (g[i], 0, 0)),
        ],
        out_specs=row_spec(_O),
    )
    return pl.pallas_call(
        _expert_body,
        grid_spec=grid_spec,
        out_shape=jax.ShapeDtypeStruct((_P, _O), jnp.bfloat16),
    )(gid, act, xs, w1r, res, w2, w3, w4, vec)


def kernel(x, gate_W, gate_b, fc1_W, fc1_b, ln1_g, ln1_b, res_W, res_b,
           fc2_W, fc2_b, ln2_g, ln2_b, fc3_W, fc3_b, ln3_g, ln3_b,
           fc4_W, fc4_b):
    logits = x @ gate_W + gate_b
    probs = jax.nn.softmax(logits, axis=-1)
    lane = jnp.arange(_E, dtype=jnp.int32)[None, :]
    i1 = jnp.argmax(probs, axis=-1).astype(jnp.int32)
    v1 = jnp.max(probs, axis=-1)
    p2 = jnp.where(lane == i1[:, None], -1.0, probs)
    i2 = jnp.argmax(p2, axis=-1).astype(jnp.int32)
    v2 = jnp.max(p2, axis=-1)
    s = v1 + v2 + 1e-9
    wn = jnp.stack([v1 / s, v2 / s], axis=1)

    e_flat = jnp.stack([i1, i2], axis=1).reshape(-1)
    onehot = (e_flat[:, None] == lane).astype(jnp.float32)
    ohb = onehot.reshape(_N * _K // 128, 128, _E)
    r_iota = jnp.arange(128, dtype=jnp.int32)
    tril_s = (r_iota[:, None] > r_iota[None, :]).astype(jnp.float32)
    pre = jnp.einsum('rs,bse->bre', tril_s, ohb)
    blocksum = jnp.sum(ohb, axis=1)
    blockpre = jnp.cumsum(blocksum, axis=0) - blocksum
    rank_full = (blockpre[:, None, :] + pre).reshape(_N * _K, _E)
    rank = jnp.take_along_axis(rank_full, e_flat[:, None], axis=1)[:, 0]
    rank = rank.astype(jnp.int32)
    counts = jnp.sum(blocksum, axis=0).astype(jnp.int32)

    padded = ((counts + _T - 1) // _T) * _T
    pad_end = jnp.cumsum(padded)
    pad_off = pad_end - padded
    dest = (pad_off[e_flat] + rank).astype(jnp.int32)

    tok = (jnp.arange(_N * _K, dtype=jnp.int32) // _K)
    gather_idx = jnp.zeros((_P,), jnp.int32).at[dest].set(tok)
    pos = dest.reshape(_N, _K)

    tile_start = jnp.arange(_P // _T, dtype=jnp.int32) * _T
    gid = jnp.searchsorted(pad_end, tile_start, side='right')
    gid = jnp.minimum(gid, _E - 1).astype(jnp.int32)
    act = (tile_start < pad_end[_E - 1]).astype(jnp.int32)

    xs = jnp.take(x.astype(jnp.bfloat16), gather_idx, axis=0)

    vec = jnp.concatenate(
        [fc1_b, res_b, ln1_g, ln1_b, fc2_b, ln2_g, ln2_b,
         fc3_b, ln3_g, ln3_b, fc4_b], axis=1)[:, None, :]

    out_sorted = _grouped_mlp(
        gid, act, xs, fc1_W, res_W, fc2_W, fc3_W, fc4_W, vec)

    y = (jnp.take(out_sorted, pos[:, 0], axis=0) * wn[:, 0:1]
         + jnp.take(out_sorted, pos[:, 1], axis=0) * wn[:, 1:2])
    return y


# speedup vs baseline: 1.5494x; 1.0973x over previous
import functools

import jax
import jax.numpy as jnp
from jax.experimental import pallas as pl
from jax.experimental.pallas import tpu as pltpu

_E = 8
_K = 2
_D = 1024
_H = 1024
_F = 512
_O = 1024
_N = 2048
_T = 256
_P = _N * _K + _E * _T
_V = 9728


def _ln(t, g, b):
    m = jnp.mean(t, axis=-1, keepdims=True)
    v = jnp.mean((t - m) ** 2, axis=-1, keepdims=True)
    return (t - m) * jax.lax.rsqrt(v + 1e-5) * g + b


def _gelu(t):
    return 0.5 * t * (1.0 + jax.lax.erf(t * 0.7071067811865476))


def _expert_body(gid_ref, act_ref, x_ref, w1r_ref, res_ref, w2_ref, w3_ref,
                 w4_ref, vec_ref, out_ref):
    @pl.when(act_ref[pl.program_id(0)] > 0)
    def _():
        x = x_ref[...]
        v = vec_ref[0, 0]
        b16 = lambda w: w.astype(jnp.bfloat16)
        h = jnp.dot(x, b16(w1r_ref[0]), preferred_element_type=jnp.float32)
        h = h + v[0:1024]
        r = jnp.dot(x, b16(res_ref[0]), preferred_element_type=jnp.float32)
        r = r + v[1024:2048]
        h = _gelu(_ln(h, v[2048:3072], v[3072:4096])) + r
        h = h.astype(jnp.bfloat16)
        h = jnp.dot(h, b16(w2_ref[0]), preferred_element_type=jnp.float32) + v[4096:5120]
        h = _gelu(_ln(h, v[5120:6144], v[6144:7168])).astype(jnp.bfloat16)
        h = jnp.dot(h, b16(w3_ref[0]), preferred_element_type=jnp.float32) + v[7168:7680]
        h = _gelu(_ln(h, v[7680:8192], v[8192:8704])).astype(jnp.bfloat16)
        o = jnp.dot(h, b16(w4_ref[0]), preferred_element_type=jnp.float32) + v[8704:9728]
        out_ref[...] = o.astype(jnp.bfloat16)


def _grouped_mlp(gid, act, xs, w1r, res, w2, w3, w4, vec):
    n_tiles = _P // _T

    def row_spec(cols):
        return pl.BlockSpec((_T, cols), lambda i, g, a: (i, 0))

    def w3_spec(r, c):
        return pl.BlockSpec((1, r, c), lambda i, g, a: (g[i], 0, 0))

    grid_spec = pltpu.PrefetchScalarGridSpec(
        num_scalar_prefetch=2,
        grid=(n_tiles,),
        in_specs=[
            row_spec(_D),
            w3_spec(_D, _H),
            w3_spec(_D, _H),
            w3_spec(_H, _H),
            w3_spec(_H, _F),
            w3_spec(_F, _O),
            pl.BlockSpec((1, 1, _V), lambda i, g, a: (g[i], 0, 0)),
        ],
        out_specs=row_spec(_O),
    )
    return pl.pallas_call(
        _expert_body,
        grid_spec=grid_spec,
        out_shape=jax.ShapeDtypeStruct((_P, _O), jnp.bfloat16),
    )(gid, act, xs, w1r, res, w2, w3, w4, vec)


def kernel(x, gate_W, gate_b, fc1_W, fc1_b, ln1_g, ln1_b, res_W, res_b,
           fc2_W, fc2_b, ln2_g, ln2_b, fc3_W, fc3_b, ln3_g, ln3_b,
           fc4_W, fc4_b):
    logits = x @ gate_W + gate_b
    lane = jnp.arange(_E, dtype=jnp.int32)[None, :]
    i1 = jnp.argmax(logits, axis=-1).astype(jnp.int32)
    l1 = jnp.max(logits, axis=-1)
    m2 = jnp.where(lane == i1[:, None], -jnp.inf, logits)
    i2 = jnp.argmax(m2, axis=-1).astype(jnp.int32)
    l2 = jnp.max(m2, axis=-1)
    w1 = 1.0 / (1.0 + jnp.exp(l2 - l1))
    wn = jnp.stack([w1, 1.0 - w1], axis=1)

    e_flat = jnp.stack([i1, i2], axis=1).reshape(-1)
    onehot = (e_flat[:, None] == lane).astype(jnp.float32)
    ohb = onehot.reshape(_N * _K // 128, 128, _E)
    r_iota = jnp.arange(128, dtype=jnp.int32)
    tril_s = (r_iota[:, None] > r_iota[None, :]).astype(jnp.float32)
    pre = jnp.einsum('rs,bse->bre', tril_s, ohb)
    blocksum = jnp.sum(ohb, axis=1)
    blockpre = jnp.cumsum(blocksum, axis=0) - blocksum
    rank_full = (blockpre[:, None, :] + pre).reshape(_N * _K, _E)
    rank = jnp.sum(rank_full * onehot, axis=1).astype(jnp.int32)
    counts = jnp.sum(blocksum, axis=0).astype(jnp.int32)

    padded = ((counts + _T - 1) // _T) * _T
    pad_end = jnp.cumsum(padded)
    pad_off = pad_end - padded
    dest = (pad_off[e_flat] + rank).astype(jnp.int32)

    tok = (jnp.arange(_N * _K, dtype=jnp.int32) // _K)
    gather_idx = jnp.zeros((_P,), jnp.int32).at[dest].set(tok)
    pos = dest.reshape(_N, _K)

    tile_start = jnp.arange(_P // _T, dtype=jnp.int32) * _T
    gid = jnp.searchsorted(pad_end, tile_start, side='right')
    gid = jnp.minimum(gid, _E - 1).astype(jnp.int32)
    act = (tile_start < pad_end[_E - 1]).astype(jnp.int32)

    xs = jnp.take(x.astype(jnp.bfloat16), gather_idx, axis=0)

    vec = jnp.concatenate(
        [fc1_b, res_b, ln1_g, ln1_b, fc2_b, ln2_g, ln2_b,
         fc3_b, ln3_g, ln3_b, fc4_b], axis=1)[:, None, :]

    out_sorted = _grouped_mlp(
        gid, act, xs, fc1_W, res_W, fc2_W, fc3_W, fc4_W, vec)

    y = (jnp.take(out_sorted, pos[:, 0], axis=0) * wn[:, 0:1]
         + jnp.take(out_sorted, pos[:, 1], axis=0) * wn[:, 1:2])
    return y
